# baseline (speedup 1.0000x reference)
import jax
import jax.numpy as jnp
from jax.experimental import pallas as pl
from jax.experimental.pallas import tpu as pltpu

_M = 1000000
_D = 64
_B = 16384
_COPY_ROWS = 10000


def _copy_body(x_ref, o_ref):
    o_ref[...] = x_ref[...]


def _scatter_body(idx_ref, u_ref, base_ref, o_ref):
    del idx_ref, base_ref
    o_ref[...] = u_ref[...]


def kernel(inputs, indices, updates):
    idx = indices[:, 0].astype(jnp.int32)

    copied = pl.pallas_call(
        _copy_body,
        grid=(_M // _COPY_ROWS,),
        in_specs=[pl.BlockSpec((_COPY_ROWS, _D), lambda i: (i, 0))],
        out_specs=pl.BlockSpec((_COPY_ROWS, _D), lambda i: (i, 0)),
        out_shape=jax.ShapeDtypeStruct((_M, _D), jnp.float32),
    )(inputs)

    out = pl.pallas_call(
        _scatter_body,
        grid_spec=pltpu.PrefetchScalarGridSpec(
            num_scalar_prefetch=1,
            grid=(_B,),
            in_specs=[
                pl.BlockSpec((1, 1, _D), lambda b, idx_ref: (b, 0, 0)),
                pl.BlockSpec(memory_space=pl.ANY),
            ],
            out_specs=pl.BlockSpec(
                (1, 1, _D), lambda b, idx_ref: (idx_ref[b], 0, 0)),
        ),
        out_shape=jax.ShapeDtypeStruct((_M, 1, _D), jnp.float32),
        input_output_aliases={2: 0},
        compiler_params=pltpu.CompilerParams(
            dimension_semantics=("arbitrary",)),
    )(idx, updates.reshape(_B, 1, _D), copied.reshape(_M, 1, _D))
    return out.reshape(_M, _D)
